# Initial kernel scaffold; baseline (speedup 1.0000x reference)
import jax
import jax.numpy as jnp
from jax.experimental import pallas as pl


def kernel(x, edge_index, W1, b1, gamma1, beta1, W2, b2, gamma2, beta2, Wc1, bc1, Wc2, bc2):
    raise NotImplementedError("write your pallas kernel here")


# speedup vs baseline: 10.5512x; 10.5512x over previous
import functools

import jax
import jax.numpy as jnp
from jax import lax
from jax.experimental import pallas as pl
from jax.experimental.pallas import tpu as pltpu
from jax.experimental.pallas import tpu_sc as plsc

N_NODES = 10000
N_EDGES = 320000
D_IN = 128
D_HID = 128
D_OUT = 64

NC = 2
NS = 16
NW = NC * NS
NPAD = 10240
ROWS_PER_TILE = NPAD // NS
DUMMY = N_NODES
CH = 128
NCHUNK = 79
EPW = NCHUNK * CH
EPAD = EPW * NW

_mesh = plsc.VectorSubcoreMesh(core_axis_name="c", subcore_axis_name="s")


@functools.partial(
    pl.kernel,
    out_type=jax.ShapeDtypeStruct((NW, NPAD), jnp.float32),
    mesh=_mesh,
    scratch_types=[
        pltpu.VMEM((EPW,), jnp.int32),
        pltpu.VMEM((NPAD,), jnp.float32),
    ],
    compiler_params=pltpu.CompilerParams(needs_layout_passes=False),
)
def _deg_kernel(dst_hbm, zeros_hbm, out_hbm, idx_v, hist_v):
    cid = lax.axis_index("c")
    sid = lax.axis_index("s")
    wid = cid * NS + sid
    pltpu.sync_copy(zeros_hbm, hist_v)
    pltpu.sync_copy(dst_hbm.at[wid], idx_v)
    ones16 = jnp.ones((16,), jnp.float32)

    def body(j, _):
        iv = idx_v[pl.ds(j * 16, 16)]
        plsc.addupdate_scatter(hist_v, [iv], ones16)
        return 0

    lax.fori_loop(0, EPW // 16, body, 0)
    pltpu.sync_copy(hist_v, out_hbm.at[wid])


@functools.partial(
    pl.kernel,
    out_type=jax.ShapeDtypeStruct((NC, NPAD, D_HID), jnp.float32),
    mesh=_mesh,
    scratch_types=[
        pltpu.VMEM((CH,), jnp.int32),
        pltpu.VMEM((CH,), jnp.int32),
        pltpu.VMEM((CH, D_HID), jnp.float32),
        pltpu.VMEM_SHARED((NPAD, D_HID), jnp.float32),
        pltpu.SemaphoreType.DMA,
    ],
)
def _prop_kernel(h_hbm, src_hbm, dst_hbm, zeros_hbm, out_hbm,
                 src_v, dst_v, rows_v, acc_sh, sem):
    cid = lax.axis_index("c")
    sid = lax.axis_index("s")
    wid = cid * NS + sid
    base = sid * ROWS_PER_TILE
    pltpu.sync_copy(zeros_hbm.at[pl.ds(base, ROWS_PER_TILE)],
                    acc_sh.at[pl.ds(base, ROWS_PER_TILE)])
    plsc.subcore_barrier()

    def body(j, _):
        pltpu.sync_copy(src_hbm.at[wid, j], src_v)
        pltpu.sync_copy(dst_hbm.at[wid, j], dst_v)
        pltpu.async_copy(h_hbm.at[src_v], rows_v, sem).wait()
        pltpu.sync_copy(rows_v, acc_sh.at[dst_v], add=True)
        return 0

    lax.fori_loop(0, NCHUNK, body, 0)
    plsc.subcore_barrier()
    pltpu.sync_copy(acc_sh.at[pl.ds(base, ROWS_PER_TILE)],
                    out_hbm.at[cid, pl.ds(base, ROWS_PER_TILE)])


_BLK = 2048
_GRID = NPAD // _BLK
_BN_SCALE = 1.0 / (1.0 + 1e-5) ** 0.5


def _dis_from_deg(deg_blk):
    ones_col = jnp.ones((NW, 1), jnp.float32)
    s = lax.dot_general(deg_blk, ones_col, (((0,), (0,)), ((), ())),
                        preferred_element_type=jnp.float32)
    return lax.rsqrt(s + 1.0)


def _tc1_body(x_ref, deg_ref, w1_ref, o_ref):
    dis = _dis_from_deg(deg_ref[...])
    xc = jnp.nan_to_num(x_ref[...], nan=0.0)
    h = jnp.dot(xc, w1_ref[...], preferred_element_type=jnp.float32)
    o_ref[...] = h * dis


def _tc2_body(acc_ref, hp_ref, deg_ref, b_ref, g_ref, be_ref, w2_ref, o_ref):
    dis = _dis_from_deg(deg_ref[...])
    t = dis * (acc_ref[0] + acc_ref[1] + hp_ref[...]) + b_ref[...][None, :]
    t = t * (g_ref[...] * _BN_SCALE)[None, :] + be_ref[...][None, :]
    t = jnp.maximum(t, 0.0)
    o_ref[...] = jnp.dot(t, w2_ref[...], preferred_element_type=jnp.float32) * dis


def _tc3_body(acc_ref, hp_ref, deg_ref, b_ref, g_ref, be_ref,
              wc1_ref, bc1_ref, wc2_ref, bc2_ref, o_ref, colsum):
    i = pl.program_id(0)
    dis = _dis_from_deg(deg_ref[...])
    t = dis * (acc_ref[0] + acc_ref[1] + hp_ref[...]) + b_ref[...][None, :]
    t = t * (g_ref[...] * _BN_SCALE)[None, :] + be_ref[...][None, :]
    t = jnp.maximum(t, 0.0)
    row = i * _BLK + lax.broadcasted_iota(jnp.int32, (_BLK, 1), 0)
    t = jnp.where(row < N_NODES, t, 0.0)
    part = jnp.sum(t, axis=0, keepdims=True)

    @pl.when(i == 0)
    def _():
        colsum[...] = part

    @pl.when(i > 0)
    def _():
        colsum[...] = colsum[...] + part

    @pl.when(i == _GRID - 1)
    def _():
        g = colsum[...] * (1.0 / N_NODES)
        u = jnp.dot(g, wc1_ref[...], preferred_element_type=jnp.float32)
        u = jnp.maximum(u + bc1_ref[...][None, :], 0.0)
        v = jnp.dot(u, wc2_ref[...], preferred_element_type=jnp.float32)
        o_ref[...] = v + bc2_ref[...][None, :]


def _rows_spec():
    return pl.BlockSpec((_BLK, D_HID), lambda i: (i, 0))


def _acc_spec():
    return pl.BlockSpec((NC, _BLK, D_HID), lambda i: (0, i, 0))


def _deg_spec():
    return pl.BlockSpec((NW, _BLK), lambda i: (0, i))


def _full_spec(*shape):
    return pl.BlockSpec(shape, lambda i: tuple(0 for _ in shape))


def kernel(x, edge_index, W1, b1, gamma1, beta1, W2, b2, gamma2, beta2,
           Wc1, bc1, Wc2, bc2):
    x = x.astype(jnp.float32)
    src = edge_index[0].astype(jnp.int32)
    dst = edge_index[1].astype(jnp.int32)
    pad = EPAD - N_EDGES
    src = jnp.concatenate([src, jnp.zeros((pad,), jnp.int32)])
    dst = jnp.concatenate([dst, jnp.full((pad,), DUMMY, jnp.int32)])
    src3 = src.reshape(NW, NCHUNK, CH)
    dst3 = dst.reshape(NW, NCHUNK, CH)
    dst2 = dst.reshape(NW, EPW)
    xp = jnp.zeros((NPAD, D_IN), jnp.float32).at[:N_NODES].set(x)
    zeros1 = jnp.zeros((NPAD,), jnp.float32)
    zeros2 = jnp.zeros((NPAD, D_HID), jnp.float32)

    deg = _deg_kernel(dst2, zeros1)

    h1p = pl.pallas_call(
        _tc1_body,
        grid=(_GRID,),
        in_specs=[_rows_spec(), _deg_spec(), _full_spec(D_IN, D_HID)],
        out_specs=_rows_spec(),
        out_shape=jax.ShapeDtypeStruct((NPAD, D_HID), jnp.float32),
    )(xp, deg, W1)

    acc1 = _prop_kernel(h1p, src3, dst3, zeros2)

    h2p = pl.pallas_call(
        _tc2_body,
        grid=(_GRID,),
        in_specs=[_acc_spec(), _rows_spec(), _deg_spec(),
                  _full_spec(D_HID), _full_spec(D_HID), _full_spec(D_HID),
                  _full_spec(D_HID, D_HID)],
        out_specs=_rows_spec(),
        out_shape=jax.ShapeDtypeStruct((NPAD, D_HID), jnp.float32),
    )(acc1, h1p, deg, b1, gamma1, beta1, W2)

    acc2 = _prop_kernel(h2p, src3, dst3, zeros2)

    out = pl.pallas_call(
        _tc3_body,
        grid=(_GRID,),
        in_specs=[_acc_spec(), _rows_spec(), _deg_spec(),
                  _full_spec(D_HID), _full_spec(D_HID), _full_spec(D_HID),
                  _full_spec(D_HID, D_HID // 2), _full_spec(D_HID // 2),
                  _full_spec(D_HID // 2, D_OUT), _full_spec(D_OUT)],
        out_specs=_full_spec(1, D_OUT),
        out_shape=jax.ShapeDtypeStruct((1, D_OUT), jnp.float32),
        scratch_shapes=[pltpu.VMEM((1, D_HID), jnp.float32)],
    )(acc2, h2p, deg, b2, gamma2, beta2, Wc1, bc1, Wc2, bc2)

    return out
